# baseline (speedup 1.0000x reference)
import functools

import jax
import jax.numpy as jnp
from jax import lax
from jax.experimental import pallas as pl
from jax.experimental.pallas import tpu as pltpu
from jax.experimental.pallas import tpu_sc as plsc

N = 10000
F = 128
E = 320000

NTILES = 32
CHUNK = 128
NP = 10112
ROWS_PER_TILE = NP // 16
BLK = 8
NBLK_PAIR = 20
TOTAL_CHUNKS = 16 * NBLK_PAIR * BLK
EP = TOTAL_CHUNKS * CHUNK

NBLK0 = 17
NBLK1 = 3

_mesh = plsc.VectorSubcoreMesh(core_axis_name="c", subcore_axis_name="s")


def _tile_plan(c, s):
    nblk = jnp.where(c == 0, NBLK0, NBLK1)
    base = jnp.where(c == 0, s * NBLK0, 16 * NBLK0 + s * NBLK1) * BLK
    return base, nblk


def _deg_body(dst_hbm, ones_hbm, zero_hbm, out_hbm, dblk, ones_v, hist,
              sem0, sem1):
    c = lax.axis_index("c")
    s = lax.axis_index("s")
    base, nblk = _tile_plan(c, s)
    r0 = s * ROWS_PER_TILE
    pltpu.sync_copy(ones_hbm, ones_v)
    pltpu.sync_copy(zero_hbm.at[pl.ds(r0, ROWS_PER_TILE)],
                    hist.at[pl.ds(r0, ROWS_PER_TILE)])
    plsc.subcore_barrier()

    @pl.loop(0, nblk)
    def _(b):
        pltpu.sync_copy(dst_hbm.at[pl.ds(base + b * BLK, BLK)], dblk)
        for j in range(BLK):
            sem = sem0 if j % 2 == 0 else sem1
            if j >= 2:
                pltpu.make_async_copy(ones_v, hist.at[dblk.at[j - 2]],
                                      sem).wait()
            pltpu.async_copy(ones_v, hist.at[dblk.at[j]], sem, add=True)
        pltpu.make_async_copy(ones_v, hist.at[dblk.at[BLK - 2]], sem0).wait()
        pltpu.make_async_copy(ones_v, hist.at[dblk.at[BLK - 1]], sem1).wait()

    plsc.subcore_barrier()
    pltpu.sync_copy(hist.at[pl.ds(r0, ROWS_PER_TILE)],
                    out_hbm.at[c, pl.ds(r0, ROWS_PER_TILE)])


_deg_call = pl.kernel(
    _deg_body,
    out_type=jax.ShapeDtypeStruct((2, NP, F), jnp.float32),
    mesh=_mesh,
    scratch_types=[
        pltpu.VMEM((BLK, CHUNK), jnp.int32),
        pltpu.VMEM((CHUNK, F), jnp.float32),
        pltpu.VMEM_SHARED((NP, F), jnp.float32),
        pltpu.SemaphoreType.DMA,
        pltpu.SemaphoreType.DMA,
    ],
)


def _conv_body(g_hbm, src_hbm, dst_hbm, zero_hbm, out_hbm,
               sblk, dblk, rows0, rows1, acc, gsem0, gsem1):
    c = lax.axis_index("c")
    s = lax.axis_index("s")
    base, nblk = _tile_plan(c, s)
    r0 = s * ROWS_PER_TILE
    pltpu.sync_copy(zero_hbm.at[pl.ds(r0, ROWS_PER_TILE)],
                    acc.at[pl.ds(r0, ROWS_PER_TILE)])
    plsc.subcore_barrier()

    @pl.loop(0, nblk)
    def _(b):
        pltpu.sync_copy(src_hbm.at[pl.ds(base + b * BLK, BLK)], sblk)
        pltpu.sync_copy(dst_hbm.at[pl.ds(base + b * BLK, BLK)], dblk)
        pltpu.async_copy(g_hbm.at[sblk.at[0]], rows0, gsem0)
        for j in range(BLK):
            rows, gsem = (rows0, gsem0) if j % 2 == 0 else (rows1, gsem1)
            if j + 1 < BLK:
                nrows, ngsem = (rows0, gsem0) if j % 2 == 1 else (rows1, gsem1)
                pltpu.async_copy(g_hbm.at[sblk.at[j + 1]], nrows, ngsem)
            pltpu.make_async_copy(g_hbm.at[sblk.at[j]], rows, gsem).wait()
            pltpu.sync_copy(rows, acc.at[dblk.at[j]], add=True)

    plsc.subcore_barrier()
    pltpu.sync_copy(acc.at[pl.ds(r0, ROWS_PER_TILE)],
                    out_hbm.at[c, pl.ds(r0, ROWS_PER_TILE)])


_conv_call = pl.kernel(
    _conv_body,
    out_type=jax.ShapeDtypeStruct((2, NP, F), jnp.float32),
    mesh=_mesh,
    scratch_types=[
        pltpu.VMEM((BLK, CHUNK), jnp.int32),
        pltpu.VMEM((BLK, CHUNK), jnp.int32),
        pltpu.VMEM((CHUNK, F), jnp.float32),
        pltpu.VMEM((CHUNK, F), jnp.float32),
        pltpu.VMEM_SHARED((NP, F), jnp.float32),
        pltpu.SemaphoreType.DMA,
        pltpu.SemaphoreType.DMA,
    ],
)


def _dis_from(degp_ref):
    deg = degp_ref[0] + degp_ref[1] + 1.0
    return lax.rsqrt(jnp.maximum(deg, 1e-12))


def _tc_in(x_ref, w_ref, b_ref, o_ref):
    o_ref[...] = (jnp.dot(x_ref[...], w_ref[...],
                          preferred_element_type=jnp.float32) + b_ref[...])


def _tc_pre(degp_ref, h_ref, w_ref, o_ref):
    dis = _dis_from(degp_ref)
    hw = jnp.dot(h_ref[...], w_ref[...], preferred_element_type=jnp.float32)
    o_ref[...] = hw * dis


def _tc_mid(degp_ref, p_ref, g_ref, w_ref, bprev_ref, o_ref):
    dis = _dis_from(degp_ref)
    acc = p_ref[0] + p_ref[1] + g_ref[...]
    h = jnp.maximum(acc * dis + bprev_ref[...], 0.0)
    hw = jnp.dot(h, w_ref[...], preferred_element_type=jnp.float32)
    o_ref[...] = hw * dis


def _tc_out(degp_ref, p_ref, g_ref, w_ref, bprev_ref, b_ref, o_ref):
    dis = _dis_from(degp_ref)
    acc = p_ref[0] + p_ref[1] + g_ref[...]
    h = jnp.maximum(acc * dis + bprev_ref[...], 0.0)
    out = jnp.dot(h, w_ref[...], preferred_element_type=jnp.float32) + b_ref[...]
    o_ref[...] = out[:N]


_f32 = jnp.float32
_tc_in_call = pl.pallas_call(
    _tc_in, out_shape=jax.ShapeDtypeStruct((NP, F), _f32))
_tc_pre_call = pl.pallas_call(
    _tc_pre, out_shape=jax.ShapeDtypeStruct((NP, F), _f32))
_tc_mid_call = pl.pallas_call(
    _tc_mid, out_shape=jax.ShapeDtypeStruct((NP, F), _f32))
_tc_out_call = pl.pallas_call(
    _tc_out, out_shape=jax.ShapeDtypeStruct((N, F), _f32))


def kernel(x, edge_index, W_in, b_in, W0, b0, W1, b1, W_out, b_out):
    src = edge_index[0]
    dst = edge_index[1]
    pad_e = EP - E
    src_p = jnp.concatenate([src, jnp.zeros((pad_e,), jnp.int32)])
    dst_p = jnp.concatenate([dst, jnp.full((pad_e,), N, jnp.int32)])
    src2 = src_p.reshape(TOTAL_CHUNKS, CHUNK)
    dst2 = dst_p.reshape(TOTAL_CHUNKS, CHUNK)
    x_p = jnp.pad(x, ((0, NP - N), (0, 0)))
    z128 = jnp.zeros((NP, F), _f32)
    ones128 = jnp.ones((CHUNK, F), _f32)

    degp = _deg_call(dst2, ones128, z128)
    h = _tc_in_call(x_p, W_in, b_in.reshape(1, F))
    g0 = _tc_pre_call(degp, h, W0)
    p0 = _conv_call(g0, src2, dst2, z128)
    g1 = _tc_mid_call(degp, p0, g0, W1, b0.reshape(1, F))
    p1 = _conv_call(g1, src2, dst2, z128)
    return _tc_out_call(degp, p1, g1, W_out,
                        b1.reshape(1, F), b_out.reshape(1, F))


# speedup vs baseline: 1.1002x; 1.1002x over previous
import functools

import jax
import jax.numpy as jnp
from jax import lax
from jax.experimental import pallas as pl
from jax.experimental.pallas import tpu as pltpu
from jax.experimental.pallas import tpu_sc as plsc

N = 10000
F = 128
E = 320000

NTILES = 32
CHUNK = 128
NP = 10112
ROWS_PER_TILE = NP // 16
BLK = 8
NBLK_PAIR = 20
TOTAL_CHUNKS = 16 * NBLK_PAIR * BLK
EP = TOTAL_CHUNKS * CHUNK

NBLK0 = 19
NBLK1 = 1

_mesh = plsc.VectorSubcoreMesh(core_axis_name="c", subcore_axis_name="s")


def _tile_plan(c, s):
    nblk = jnp.where(c == 0, NBLK0, NBLK1)
    base = jnp.where(c == 0, s * NBLK0, 16 * NBLK0 + s * NBLK1) * BLK
    return base, nblk


def _deg_body(dst_hbm, ones_hbm, zero_hbm, out_hbm, dblk, ones_v, hist,
              sem0, sem1):
    c = lax.axis_index("c")
    s = lax.axis_index("s")
    base, nblk = _tile_plan(c, s)
    r0 = s * ROWS_PER_TILE
    pltpu.sync_copy(ones_hbm, ones_v)
    pltpu.sync_copy(zero_hbm.at[pl.ds(r0, ROWS_PER_TILE)],
                    hist.at[pl.ds(r0, ROWS_PER_TILE)])
    plsc.subcore_barrier()

    @pl.loop(0, nblk)
    def _(b):
        pltpu.sync_copy(dst_hbm.at[pl.ds(base + b * BLK, BLK)], dblk)
        for j in range(BLK):
            sem = sem0 if j % 2 == 0 else sem1
            if j >= 2:
                pltpu.make_async_copy(ones_v, hist.at[dblk.at[j - 2]],
                                      sem).wait()
            pltpu.async_copy(ones_v, hist.at[dblk.at[j]], sem, add=True)
        pltpu.make_async_copy(ones_v, hist.at[dblk.at[BLK - 2]], sem0).wait()
        pltpu.make_async_copy(ones_v, hist.at[dblk.at[BLK - 1]], sem1).wait()

    plsc.subcore_barrier()
    pltpu.sync_copy(hist.at[pl.ds(r0, ROWS_PER_TILE)],
                    out_hbm.at[c, pl.ds(r0, ROWS_PER_TILE)])


_deg_call = pl.kernel(
    _deg_body,
    out_type=jax.ShapeDtypeStruct((2, NP, F), jnp.float32),
    mesh=_mesh,
    scratch_types=[
        pltpu.VMEM((BLK, CHUNK), jnp.int32),
        pltpu.VMEM((CHUNK, F), jnp.float32),
        pltpu.VMEM_SHARED((NP, F), jnp.float32),
        pltpu.SemaphoreType.DMA,
        pltpu.SemaphoreType.DMA,
    ],
)


def _conv_body(g_hbm, src_hbm, dst_hbm, zero_hbm, out_hbm,
               sblk, dblk, rows0, rows1, acc, gsem0, gsem1):
    c = lax.axis_index("c")
    s = lax.axis_index("s")
    base, nblk = _tile_plan(c, s)
    r0 = s * ROWS_PER_TILE
    pltpu.sync_copy(zero_hbm.at[pl.ds(r0, ROWS_PER_TILE)],
                    acc.at[pl.ds(r0, ROWS_PER_TILE)])
    plsc.subcore_barrier()

    @pl.loop(0, nblk)
    def _(b):
        pltpu.sync_copy(src_hbm.at[pl.ds(base + b * BLK, BLK)], sblk)
        pltpu.sync_copy(dst_hbm.at[pl.ds(base + b * BLK, BLK)], dblk)
        pltpu.async_copy(g_hbm.at[sblk.at[0]], rows0, gsem0)
        for j in range(BLK):
            rows, gsem = (rows0, gsem0) if j % 2 == 0 else (rows1, gsem1)
            if j + 1 < BLK:
                nrows, ngsem = (rows0, gsem0) if j % 2 == 1 else (rows1, gsem1)
                pltpu.async_copy(g_hbm.at[sblk.at[j + 1]], nrows, ngsem)
            pltpu.make_async_copy(g_hbm.at[sblk.at[j]], rows, gsem).wait()
            pltpu.sync_copy(rows, acc.at[dblk.at[j]], add=True)

    plsc.subcore_barrier()
    pltpu.sync_copy(acc.at[pl.ds(r0, ROWS_PER_TILE)],
                    out_hbm.at[c, pl.ds(r0, ROWS_PER_TILE)])


_conv_call = pl.kernel(
    _conv_body,
    out_type=jax.ShapeDtypeStruct((2, NP, F), jnp.float32),
    mesh=_mesh,
    scratch_types=[
        pltpu.VMEM((BLK, CHUNK), jnp.int32),
        pltpu.VMEM((BLK, CHUNK), jnp.int32),
        pltpu.VMEM((CHUNK, F), jnp.float32),
        pltpu.VMEM((CHUNK, F), jnp.float32),
        pltpu.VMEM_SHARED((NP, F), jnp.float32),
        pltpu.SemaphoreType.DMA,
        pltpu.SemaphoreType.DMA,
    ],
)


def _dis_from(degp_ref):
    deg = degp_ref[0] + degp_ref[1] + 1.0
    return lax.rsqrt(jnp.maximum(deg, 1e-12))


def _tc_in(x_ref, w_ref, b_ref, o_ref):
    o_ref[...] = (jnp.dot(x_ref[...], w_ref[...],
                          preferred_element_type=jnp.float32) + b_ref[...])


def _tc_pre(degp_ref, h_ref, w_ref, o_ref):
    dis = _dis_from(degp_ref)
    hw = jnp.dot(h_ref[...], w_ref[...], preferred_element_type=jnp.float32)
    o_ref[...] = hw * dis


def _tc_mid(degp_ref, p_ref, g_ref, w_ref, bprev_ref, o_ref):
    dis = _dis_from(degp_ref)
    acc = p_ref[0] + p_ref[1] + g_ref[...]
    h = jnp.maximum(acc * dis + bprev_ref[...], 0.0)
    hw = jnp.dot(h, w_ref[...], preferred_element_type=jnp.float32)
    o_ref[...] = hw * dis


def _tc_out(degp_ref, p_ref, g_ref, w_ref, bprev_ref, b_ref, o_ref):
    dis = _dis_from(degp_ref)
    acc = p_ref[0] + p_ref[1] + g_ref[...]
    h = jnp.maximum(acc * dis + bprev_ref[...], 0.0)
    out = jnp.dot(h, w_ref[...], preferred_element_type=jnp.float32) + b_ref[...]
    o_ref[...] = out[:N]


_f32 = jnp.float32
_tc_in_call = pl.pallas_call(
    _tc_in, out_shape=jax.ShapeDtypeStruct((NP, F), _f32))
_tc_pre_call = pl.pallas_call(
    _tc_pre, out_shape=jax.ShapeDtypeStruct((NP, F), _f32))
_tc_mid_call = pl.pallas_call(
    _tc_mid, out_shape=jax.ShapeDtypeStruct((NP, F), _f32))
_tc_out_call = pl.pallas_call(
    _tc_out, out_shape=jax.ShapeDtypeStruct((N, F), _f32))


def kernel(x, edge_index, W_in, b_in, W0, b0, W1, b1, W_out, b_out):
    src = edge_index[0]
    dst = edge_index[1]
    pad_e = EP - E
    src_p = jnp.concatenate([src, jnp.zeros((pad_e,), jnp.int32)])
    dst_p = jnp.concatenate([dst, jnp.full((pad_e,), N, jnp.int32)])
    src2 = src_p.reshape(TOTAL_CHUNKS, CHUNK)
    dst2 = dst_p.reshape(TOTAL_CHUNKS, CHUNK)
    x_p = jnp.pad(x, ((0, NP - N), (0, 0)))
    z128 = jnp.zeros((NP, F), _f32)
    ones128 = jnp.ones((CHUNK, F), _f32)

    degp = _deg_call(dst2, ones128, z128)
    h = _tc_in_call(x_p, W_in, b_in.reshape(1, F))
    g0 = _tc_pre_call(degp, h, W0)
    p0 = _conv_call(g0, src2, dst2, z128)
    g1 = _tc_mid_call(degp, p0, g0, W1, b0.reshape(1, F))
    p1 = _conv_call(g1, src2, dst2, z128)
    return _tc_out_call(degp, p1, g1, W_out,
                        b1.reshape(1, F), b_out.reshape(1, F))


# speedup vs baseline: 1.1131x; 1.0117x over previous
import functools

import jax
import jax.numpy as jnp
from jax import lax
from jax.experimental import pallas as pl
from jax.experimental.pallas import tpu as pltpu
from jax.experimental.pallas import tpu_sc as plsc

N = 10000
F = 128
E = 320000

NTILES = 32
CHUNK = 128
NP = 10112
ROWS_PER_TILE = NP // 16
BLK = 8
NBLK_PAIR = 20
TOTAL_CHUNKS = 16 * NBLK_PAIR * BLK
EP = TOTAL_CHUNKS * CHUNK

NBLK0 = 19
NBLK1 = 1

_mesh = plsc.VectorSubcoreMesh(core_axis_name="c", subcore_axis_name="s")


def _tile_plan(c, s):
    nblk = jnp.where(c == 0, NBLK0, NBLK1)
    base = jnp.where(c == 0, s * NBLK0, 16 * NBLK0 + s * NBLK1) * BLK
    return base, nblk


def _deg_plan(c, s):
    nblk = jnp.where(c == 0, 0, NBLK_PAIR)
    return s * NBLK_PAIR * BLK, nblk


def _deg_body(dst_hbm, ones_hbm, zero_hbm, out_hbm, dblk, ones_v, hist,
              sem0, sem1):
    c = lax.axis_index("c")
    s = lax.axis_index("s")
    base, nblk = _deg_plan(c, s)
    r0 = s * ROWS_PER_TILE

    @pl.when(c == 1)
    def _():
        pltpu.sync_copy(ones_hbm, ones_v)
        pltpu.sync_copy(zero_hbm.at[pl.ds(r0, ROWS_PER_TILE)],
                        hist.at[pl.ds(r0, ROWS_PER_TILE)])

    plsc.subcore_barrier()

    @pl.loop(0, nblk)
    def _(b):
        pltpu.sync_copy(dst_hbm.at[pl.ds(base + b * BLK, BLK)], dblk)
        for j in range(BLK):
            sem = sem0 if j % 2 == 0 else sem1
            if j >= 2:
                pltpu.make_async_copy(ones_v, hist.at[dblk.at[j - 2]],
                                      sem).wait()
            pltpu.async_copy(ones_v, hist.at[dblk.at[j]], sem, add=True)
        pltpu.make_async_copy(ones_v, hist.at[dblk.at[BLK - 2]], sem0).wait()
        pltpu.make_async_copy(ones_v, hist.at[dblk.at[BLK - 1]], sem1).wait()

    plsc.subcore_barrier()

    @pl.when(c == 1)
    def _():
        pltpu.sync_copy(hist.at[pl.ds(r0, ROWS_PER_TILE)],
                        out_hbm.at[pl.ds(r0, ROWS_PER_TILE)])


_deg_call = pl.kernel(
    _deg_body,
    out_type=jax.ShapeDtypeStruct((NP, F), jnp.float32),
    mesh=_mesh,
    scratch_types=[
        pltpu.VMEM((BLK, CHUNK), jnp.int32),
        pltpu.VMEM((CHUNK, F), jnp.float32),
        pltpu.VMEM_SHARED((NP, F), jnp.float32),
        pltpu.SemaphoreType.DMA,
        pltpu.SemaphoreType.DMA,
    ],
)


def _conv_body(g_hbm, src_hbm, dst_hbm, zero_hbm, out_hbm,
               sblk, dblk, rows0, rows1, acc, gsem0, gsem1):
    c = lax.axis_index("c")
    s = lax.axis_index("s")
    base, nblk = _tile_plan(c, s)
    r0 = s * ROWS_PER_TILE
    pltpu.sync_copy(zero_hbm.at[pl.ds(r0, ROWS_PER_TILE)],
                    acc.at[pl.ds(r0, ROWS_PER_TILE)])
    plsc.subcore_barrier()

    @pl.loop(0, nblk)
    def _(b):
        pltpu.sync_copy(src_hbm.at[pl.ds(base + b * BLK, BLK)], sblk)
        pltpu.sync_copy(dst_hbm.at[pl.ds(base + b * BLK, BLK)], dblk)
        pltpu.async_copy(g_hbm.at[sblk.at[0]], rows0, gsem0)
        for j in range(BLK):
            rows, gsem = (rows0, gsem0) if j % 2 == 0 else (rows1, gsem1)
            if j + 1 < BLK:
                nrows, ngsem = (rows0, gsem0) if j % 2 == 1 else (rows1, gsem1)
                pltpu.async_copy(g_hbm.at[sblk.at[j + 1]], nrows, ngsem)
            pltpu.make_async_copy(g_hbm.at[sblk.at[j]], rows, gsem).wait()
            pltpu.sync_copy(rows, acc.at[dblk.at[j]], add=True)

    plsc.subcore_barrier()
    pltpu.sync_copy(acc.at[pl.ds(r0, ROWS_PER_TILE)],
                    out_hbm.at[c, pl.ds(r0, ROWS_PER_TILE)])


_conv_call = pl.kernel(
    _conv_body,
    out_type=jax.ShapeDtypeStruct((2, NP, F), jnp.float32),
    mesh=_mesh,
    scratch_types=[
        pltpu.VMEM((BLK, CHUNK), jnp.int32),
        pltpu.VMEM((BLK, CHUNK), jnp.int32),
        pltpu.VMEM((CHUNK, F), jnp.float32),
        pltpu.VMEM((CHUNK, F), jnp.float32),
        pltpu.VMEM_SHARED((NP, F), jnp.float32),
        pltpu.SemaphoreType.DMA,
        pltpu.SemaphoreType.DMA,
    ],
)


def _dis_from(degp_ref):
    deg = degp_ref[...] + 1.0
    return lax.rsqrt(jnp.maximum(deg, 1e-12))


def _tc_in(x_ref, w_ref, b_ref, o_ref):
    o_ref[...] = (jnp.dot(x_ref[...], w_ref[...],
                          preferred_element_type=jnp.float32) + b_ref[...])


def _tc_pre(degp_ref, h_ref, w_ref, o_ref):
    dis = _dis_from(degp_ref)
    hw = jnp.dot(h_ref[...], w_ref[...], preferred_element_type=jnp.float32)
    o_ref[...] = hw * dis


def _tc_mid(degp_ref, p_ref, g_ref, w_ref, bprev_ref, o_ref):
    dis = _dis_from(degp_ref)
    acc = p_ref[0] + p_ref[1] + g_ref[...]
    h = jnp.maximum(acc * dis + bprev_ref[...], 0.0)
    hw = jnp.dot(h, w_ref[...], preferred_element_type=jnp.float32)
    o_ref[...] = hw * dis


def _tc_out(degp_ref, p_ref, g_ref, w_ref, bprev_ref, b_ref, o_ref):
    dis = _dis_from(degp_ref)
    acc = p_ref[0] + p_ref[1] + g_ref[...]
    h = jnp.maximum(acc * dis + bprev_ref[...], 0.0)
    out = jnp.dot(h, w_ref[...], preferred_element_type=jnp.float32) + b_ref[...]
    o_ref[...] = out[:N]


_f32 = jnp.float32
_tc_in_call = pl.pallas_call(
    _tc_in, out_shape=jax.ShapeDtypeStruct((NP, F), _f32))
_tc_pre_call = pl.pallas_call(
    _tc_pre, out_shape=jax.ShapeDtypeStruct((NP, F), _f32))
_tc_mid_call = pl.pallas_call(
    _tc_mid, out_shape=jax.ShapeDtypeStruct((NP, F), _f32))
_tc_out_call = pl.pallas_call(
    _tc_out, out_shape=jax.ShapeDtypeStruct((N, F), _f32))


def kernel(x, edge_index, W_in, b_in, W0, b0, W1, b1, W_out, b_out):
    src = edge_index[0]
    dst = edge_index[1]
    pad_e = EP - E
    src_p = jnp.concatenate([src, jnp.zeros((pad_e,), jnp.int32)])
    dst_p = jnp.concatenate([dst, jnp.full((pad_e,), N, jnp.int32)])
    src2 = src_p.reshape(TOTAL_CHUNKS, CHUNK)
    dst2 = dst_p.reshape(TOTAL_CHUNKS, CHUNK)
    x_p = jnp.pad(x, ((0, NP - N), (0, 0)))
    z128 = jnp.zeros((NP, F), _f32)
    ones128 = jnp.ones((CHUNK, F), _f32)

    degp = _deg_call(dst2, ones128, z128)
    h = _tc_in_call(x_p, W_in, b_in.reshape(1, F))
    g0 = _tc_pre_call(degp, h, W0)
    p0 = _conv_call(g0, src2, dst2, z128)
    g1 = _tc_mid_call(degp, p0, g0, W1, b0.reshape(1, F))
    p1 = _conv_call(g1, src2, dst2, z128)
    return _tc_out_call(degp, p1, g1, W_out,
                        b1.reshape(1, F), b_out.reshape(1, F))


# speedup vs baseline: 1.1623x; 1.0442x over previous
import functools

import jax
import jax.numpy as jnp
from jax import lax
from jax.experimental import pallas as pl
from jax.experimental.pallas import tpu as pltpu
from jax.experimental.pallas import tpu_sc as plsc

N = 10000
F = 128
E = 320000

NTILES = 32
CHUNK = 128
NP = 10112
ROWS_PER_TILE = NP // 16
BLK = 8
NBLK_PAIR = 20
TOTAL_CHUNKS = 16 * NBLK_PAIR * BLK
EP = TOTAL_CHUNKS * CHUNK

NBLK0 = 19
NBLK1 = 1

_mesh = plsc.VectorSubcoreMesh(core_axis_name="c", subcore_axis_name="s")


def _tile_plan(c, s):
    nblk = jnp.where(c == 0, NBLK0, NBLK1)
    base = jnp.where(c == 0, s * NBLK0, 16 * NBLK0 + s * NBLK1) * BLK
    return base, nblk


DEG0 = NBLK_PAIR // 2
DEG1 = NBLK_PAIR - DEG0


def _deg_plan(c, s):
    nblk = jnp.where(c == 0, DEG0, DEG1)
    base = jnp.where(c == 0, s * DEG0, 16 * DEG0 + s * DEG1) * BLK
    return base, nblk


def _deg_body(dst_hbm, ones_hbm, zero_hbm, out_hbm, dblk, ones_v, hist,
              sem0, sem1):
    c = lax.axis_index("c")
    s = lax.axis_index("s")
    base, nblk = _deg_plan(c, s)
    r0 = s * ROWS_PER_TILE
    pltpu.sync_copy(ones_hbm, ones_v)
    pltpu.sync_copy(zero_hbm.at[pl.ds(r0, ROWS_PER_TILE)],
                    hist.at[pl.ds(r0, ROWS_PER_TILE)])
    plsc.subcore_barrier()

    @pl.loop(0, nblk)
    def _(b):
        pltpu.sync_copy(dst_hbm.at[pl.ds(base + b * BLK, BLK)], dblk)
        for j in range(BLK):
            sem = sem0 if j % 2 == 0 else sem1
            if j >= 2:
                pltpu.make_async_copy(ones_v, hist.at[dblk.at[j - 2]],
                                      sem).wait()
            pltpu.async_copy(ones_v, hist.at[dblk.at[j]], sem, add=True)
        pltpu.make_async_copy(ones_v, hist.at[dblk.at[BLK - 2]], sem0).wait()
        pltpu.make_async_copy(ones_v, hist.at[dblk.at[BLK - 1]], sem1).wait()

    plsc.subcore_barrier()
    pltpu.sync_copy(hist.at[pl.ds(r0, ROWS_PER_TILE)],
                    out_hbm.at[c, pl.ds(r0, ROWS_PER_TILE)])


_deg_call = pl.kernel(
    _deg_body,
    out_type=jax.ShapeDtypeStruct((2, NP, F), jnp.float32),
    mesh=_mesh,
    scratch_types=[
        pltpu.VMEM((BLK, CHUNK), jnp.int32),
        pltpu.VMEM((CHUNK, F), jnp.float32),
        pltpu.VMEM_SHARED((NP, F), jnp.float32),
        pltpu.SemaphoreType.DMA,
        pltpu.SemaphoreType.DMA,
    ],
)


def _conv_body(g_hbm, src_hbm, dst_hbm, zero_hbm, out_hbm,
               sblk, dblk, rows0, rows1, acc, gsem0, gsem1):
    c = lax.axis_index("c")
    s = lax.axis_index("s")
    base, nblk = _tile_plan(c, s)
    r0 = s * ROWS_PER_TILE
    pltpu.sync_copy(zero_hbm.at[pl.ds(r0, ROWS_PER_TILE)],
                    acc.at[pl.ds(r0, ROWS_PER_TILE)])
    plsc.subcore_barrier()

    @pl.loop(0, nblk)
    def _(b):
        pltpu.sync_copy(src_hbm.at[pl.ds(base + b * BLK, BLK)], sblk)
        pltpu.sync_copy(dst_hbm.at[pl.ds(base + b * BLK, BLK)], dblk)
        pltpu.async_copy(g_hbm.at[sblk.at[0]], rows0, gsem0)
        for j in range(BLK):
            rows, gsem = (rows0, gsem0) if j % 2 == 0 else (rows1, gsem1)
            if j + 1 < BLK:
                nrows, ngsem = (rows0, gsem0) if j % 2 == 1 else (rows1, gsem1)
                pltpu.async_copy(g_hbm.at[sblk.at[j + 1]], nrows, ngsem)
            pltpu.make_async_copy(g_hbm.at[sblk.at[j]], rows, gsem).wait()
            pltpu.sync_copy(rows, acc.at[dblk.at[j]], add=True)

    plsc.subcore_barrier()
    pltpu.sync_copy(acc.at[pl.ds(r0, ROWS_PER_TILE)],
                    out_hbm.at[c, pl.ds(r0, ROWS_PER_TILE)])


_conv_call = pl.kernel(
    _conv_body,
    out_type=jax.ShapeDtypeStruct((2, NP, F), jnp.float32),
    mesh=_mesh,
    scratch_types=[
        pltpu.VMEM((BLK, CHUNK), jnp.int32),
        pltpu.VMEM((BLK, CHUNK), jnp.int32),
        pltpu.VMEM((CHUNK, F), jnp.float32),
        pltpu.VMEM((CHUNK, F), jnp.float32),
        pltpu.VMEM_SHARED((NP, F), jnp.float32),
        pltpu.SemaphoreType.DMA,
        pltpu.SemaphoreType.DMA,
    ],
)


def _dis_from(degp_ref):
    deg = degp_ref[0] + degp_ref[1] + 1.0
    return lax.rsqrt(jnp.maximum(deg, 1e-12))


def _tc_in(x_ref, w_ref, b_ref, o_ref):
    o_ref[...] = (jnp.dot(x_ref[...], w_ref[...],
                          preferred_element_type=jnp.float32) + b_ref[...])


def _tc_pre(degp_ref, h_ref, w_ref, o_ref):
    dis = _dis_from(degp_ref)
    hw = jnp.dot(h_ref[...], w_ref[...], preferred_element_type=jnp.float32)
    o_ref[...] = hw * dis


def _tc_mid(degp_ref, p_ref, g_ref, w_ref, bprev_ref, o_ref):
    dis = _dis_from(degp_ref)
    acc = p_ref[0] + p_ref[1] + g_ref[...]
    h = jnp.maximum(acc * dis + bprev_ref[...], 0.0)
    hw = jnp.dot(h, w_ref[...], preferred_element_type=jnp.float32)
    o_ref[...] = hw * dis


def _tc_out(degp_ref, p_ref, g_ref, w_ref, bprev_ref, b_ref, o_ref):
    dis = _dis_from(degp_ref)
    acc = p_ref[0] + p_ref[1] + g_ref[...]
    h = jnp.maximum(acc * dis + bprev_ref[...], 0.0)
    out = jnp.dot(h, w_ref[...], preferred_element_type=jnp.float32) + b_ref[...]
    o_ref[...] = out[:N]


_f32 = jnp.float32
_tc_in_call = pl.pallas_call(
    _tc_in, out_shape=jax.ShapeDtypeStruct((NP, F), _f32))
_tc_pre_call = pl.pallas_call(
    _tc_pre, out_shape=jax.ShapeDtypeStruct((NP, F), _f32))
_tc_mid_call = pl.pallas_call(
    _tc_mid, out_shape=jax.ShapeDtypeStruct((NP, F), _f32))
_tc_out_call = pl.pallas_call(
    _tc_out, out_shape=jax.ShapeDtypeStruct((N, F), _f32))


def kernel(x, edge_index, W_in, b_in, W0, b0, W1, b1, W_out, b_out):
    src = edge_index[0]
    dst = edge_index[1]
    pad_e = EP - E
    src_p = jnp.concatenate([src, jnp.zeros((pad_e,), jnp.int32)])
    dst_p = jnp.concatenate([dst, jnp.full((pad_e,), N, jnp.int32)])
    src2 = src_p.reshape(TOTAL_CHUNKS, CHUNK)
    dst2 = dst_p.reshape(TOTAL_CHUNKS, CHUNK)
    x_p = jnp.pad(x, ((0, NP - N), (0, 0)))
    z128 = jnp.zeros((NP, F), _f32)
    ones128 = jnp.ones((CHUNK, F), _f32)

    degp = _deg_call(dst2, ones128, z128)
    h = _tc_in_call(x_p, W_in, b_in.reshape(1, F))
    g0 = _tc_pre_call(degp, h, W0)
    p0 = _conv_call(g0, src2, dst2, z128)
    g1 = _tc_mid_call(degp, p0, g0, W1, b0.reshape(1, F))
    p1 = _conv_call(g1, src2, dst2, z128)
    return _tc_out_call(degp, p1, g1, W_out,
                        b1.reshape(1, F), b_out.reshape(1, F))
